# Initial kernel scaffold; baseline (speedup 1.0000x reference)
import jax
import jax.numpy as jnp
from jax.experimental import pallas as pl


def kernel(h_n, edge_index, W_w, W_b, Wself_w, Wself_b, Wtrans_w, Wtrans_b):
    raise NotImplementedError("write your pallas kernel here")


# speedup vs baseline: 6.5127x; 6.5127x over previous
import functools

import jax
import jax.numpy as jnp
from jax import lax
from jax.experimental import pallas as pl
from jax.experimental.pallas import tpu as pltpu
from jax.experimental.pallas import tpu_sc as plsc

N = 10000
E = 320000
D = 128

NS = 16
EPT = E // NS
K = 80
NCHUNK = EPT // K
IB = 50
NBLK = NCHUNK // IB
NPAD = 10240
RPT = NPAD // NS

BM = 400
GRID = N // BM


def _mm_body(h_ref, w_ref, ws_ref, wt_ref, b_ref, bs_ref, bt_ref,
             y_ref, hs_ref, yt_ref):
    h = h_ref[...]
    dn = (((1,), (1,)), ((), ()))
    y_ref[...] = lax.dot_general(h, w_ref[...], dn,
                                 preferred_element_type=jnp.float32) + b_ref[...]
    hs_ref[...] = lax.dot_general(h, ws_ref[...], dn,
                                  preferred_element_type=jnp.float32) + bs_ref[...]
    yt_ref[...] = lax.dot_general(h, wt_ref[...], dn,
                                  preferred_element_type=jnp.float32) + bt_ref[...]


def _matmuls(h_n, W_w, W_b, Wself_w, Wself_b, Wtrans_w, Wtrans_b):
    mspec = pl.BlockSpec((BM, D), lambda i: (i, 0))
    wspec = pl.BlockSpec((D, D), lambda i: (0, 0))
    bspec = pl.BlockSpec((1, D), lambda i: (0, 0))
    out_sds = jax.ShapeDtypeStruct((N, D), jnp.float32)
    return pl.pallas_call(
        _mm_body,
        grid=(GRID,),
        in_specs=[mspec, wspec, wspec, wspec, bspec, bspec, bspec],
        out_specs=[mspec, mspec, mspec],
        out_shape=[out_sds, out_sds, out_sds],
    )(h_n, W_w, Wself_w, Wtrans_w, W_b.reshape(1, D), Wself_b.reshape(1, D),
      Wtrans_b.reshape(1, D))


def _agg_body(y_hbm, yt_hbm, zero_hbm, srcA, dstA, srcB, dstB,
              outA, outB, src_v, dst_v, rows_v, acc_sh, sem):
    c = lax.axis_index("c")
    s = lax.axis_index("s")
    row0 = pl.multiple_of(s * RPT, 8)

    pltpu.sync_copy(zero_hbm, acc_sh.at[pl.ds(row0, RPT)])
    plsc.subcore_barrier()

    def _direction(data_hbm, src_hbm, dst_hbm):
        def blk(b, carry):
            pltpu.sync_copy(src_hbm.at[s].at[b], src_v)
            pltpu.sync_copy(dst_hbm.at[s].at[b], dst_v)

            def body(j, carry2):
                pltpu.async_copy(data_hbm.at[src_v.at[j]], rows_v, sem).wait()
                pltpu.sync_copy(rows_v, acc_sh.at[dst_v.at[j]], add=True)
                return carry2
            return lax.fori_loop(0, IB, body, carry)
        lax.fori_loop(0, NBLK, blk, 0)

    @pl.when(c == 0)
    def _():
        _direction(y_hbm, srcA, dstA)

    @pl.when(c == 1)
    def _():
        _direction(yt_hbm, srcB, dstB)

    plsc.subcore_barrier()

    @pl.when(c == 0)
    def _():
        pltpu.sync_copy(acc_sh.at[pl.ds(row0, RPT)], outA.at[pl.ds(row0, RPT)])

    @pl.when(c == 1)
    def _():
        pltpu.sync_copy(acc_sh.at[pl.ds(row0, RPT)], outB.at[pl.ds(row0, RPT)])


def _aggregate(y, yt, zeros, srcA, dstA, srcB, dstB):
    mesh = plsc.VectorSubcoreMesh(core_axis_name="c", subcore_axis_name="s")
    out_sds = jax.ShapeDtypeStruct((NPAD, D), jnp.float32)
    kern = functools.partial(
        pl.kernel,
        mesh=mesh,
        out_type=[out_sds, out_sds],
        scratch_types=[
            pltpu.VMEM((IB, K), jnp.int32),
            pltpu.VMEM((IB, K), jnp.int32),
            pltpu.VMEM((K, D), jnp.float32),
            pltpu.VMEM_SHARED((NPAD, D), jnp.float32),
            pltpu.SemaphoreType.DMA,
        ],
    )(_agg_body)
    return kern(y, yt, zeros, srcA, dstA, srcB, dstB)


def _comb_body(a_ref, b_ref, hs_ref, o_ref):
    o_ref[...] = jnp.maximum(a_ref[...] + b_ref[...] + hs_ref[...], 0.0)


def _combine(a, b, hself):
    mspec = pl.BlockSpec((BM, D), lambda i: (i, 0))
    return pl.pallas_call(
        _comb_body,
        grid=(GRID,),
        in_specs=[mspec, mspec, mspec],
        out_specs=mspec,
        out_shape=jax.ShapeDtypeStruct((N, D), jnp.float32),
    )(a, b, hself)


def kernel(h_n, edge_index, W_w, W_b, Wself_w, Wself_b, Wtrans_w, Wtrans_b):
    row = edge_index[0].astype(jnp.int32)
    col = edge_index[1].astype(jnp.int32)
    srcA = col.reshape(NS, NBLK, IB, K)
    dstA = row.reshape(NS, NBLK, IB, K)
    srcB = row.reshape(NS, NBLK, IB, K)
    dstB = col.reshape(NS, NBLK, IB, K)
    zeros = jnp.zeros((RPT, D), jnp.float32)
    y, hself, yt = _matmuls(h_n, W_w, W_b, Wself_w, Wself_b, Wtrans_w, Wtrans_b)
    acc_a, acc_b = _aggregate(y, yt, zeros, srcA, dstA, srcB, dstB)
    return _combine(acc_a, acc_b, hself)
